# baseline (speedup 1.0000x reference)
import functools

import jax
import jax.numpy as jnp
from jax import lax
from jax.experimental import pallas as pl
from jax.experimental.pallas import tpu as pltpu
from jax.experimental.pallas import tpu_sc as plsc

_B = 16384
_D = 128
_C = 1000
_NC = 2
_NS = 16
_NW = _NC * _NS
_RPW = _B // _NW
_CH = 128
_NCH = _RPW // _CH
_NBUF = 3
_LANES = 16
_JV = _D // _LANES
_CPT = 64


def _sc_body(latent_hbm, labels_hbm, centers_hbm, out_hbm,
             lab_v, buf0, buf1, buf2, stage_v, res_v, neg_sh,
             fill_sem0, fill_sem1, fill_sem2,
             add_sem0, add_sem1, add_sem2, out_sem):
    sid = lax.axis_index("s")
    wid = sid * _NC + lax.axis_index("c")
    pltpu.sync_copy(labels_hbm.at[wid], lab_v)

    base = jnp.minimum(sid * _CPT, _C - _CPT)
    pltpu.sync_copy(centers_hbm.at[pl.ds(base, _CPT)], stage_v)

    @plsc.parallel_loop(0, _CPT, 1)
    def _neg_loop(r):
        for j in range(_JV):
            stage_v[r, pl.ds(j * _LANES, _LANES)] = (
                -stage_v[r, pl.ds(j * _LANES, _LANES)])

    pltpu.sync_copy(stage_v, neg_sh.at[pl.ds(base, _CPT)])

    bufs = (buf0, buf1, buf2)
    fill_sems = (fill_sem0, fill_sem1, fill_sem2)
    add_sems = (add_sem0, add_sem1, add_sem2)

    def fill(ch):
        b = ch % _NBUF
        row0 = wid * _RPW + ch * _CH
        return pltpu.async_copy(
            latent_hbm.at[pl.ds(row0, _CH)], bufs[b], fill_sems[b])

    def gather_add(ch):
        b = ch % _NBUF
        return pltpu.async_copy(
            neg_sh.at[lab_v.at[ch]], bufs[b], add_sems[b], add=True)

    accs = tuple(jnp.zeros((_LANES,), jnp.float32) for _ in range(_JV))

    fills = {}
    adds = {}
    fills[0] = fill(0)
    fills[1] = fill(1)
    plsc.subcore_barrier()
    fills[2] = fill(2)
    fills[0].wait()
    adds[0] = gather_add(0)
    fills[1].wait()
    adds[1] = gather_add(1)
    for ch in range(_NCH):
        b = ch % _NBUF
        adds[ch].wait()
        buf = bufs[b]

        @plsc.parallel_loop(0, _CH, 1, unroll=4, carry=accs)
        def row_loop(r, acc_in):
            new = []
            for j in range(_JV):
                d = buf[r, pl.ds(j * _LANES, _LANES)]
                new.append(acc_in[j] + d * d)
            return tuple(new)

        accs = row_loop
        if ch + _NBUF < _NCH:
            fills[ch + _NBUF] = fill(ch + _NBUF)
        if ch + 2 < _NCH:
            fills[ch + 2].wait()
            adds[ch + 2] = gather_add(ch + 2)

    total = accs[0]
    for j in range(1, _JV):
        total = total + accs[j]
    res_v[...] = total
    pltpu.async_copy(res_v, out_hbm.at[wid], out_sem).wait()


@jax.jit
def _center_loss_partials(latent, labels3d, centers):
    mesh = plsc.VectorSubcoreMesh(core_axis_name="c", subcore_axis_name="s")
    run = functools.partial(
        pl.kernel,
        out_type=jax.ShapeDtypeStruct((_NW, _LANES), jnp.float32),
        mesh=mesh,
        scratch_types=[
            pltpu.VMEM((_NCH, _CH), jnp.int32),
            pltpu.VMEM((_CH, _D), jnp.float32),
            pltpu.VMEM((_CH, _D), jnp.float32),
            pltpu.VMEM((_CH, _D), jnp.float32),
            pltpu.VMEM((_CPT, _D), jnp.float32),
            pltpu.VMEM((_LANES,), jnp.float32),
            pltpu.VMEM_SHARED((_C, _D), jnp.float32),
            pltpu.SemaphoreType.DMA,
            pltpu.SemaphoreType.DMA,
            pltpu.SemaphoreType.DMA,
            pltpu.SemaphoreType.DMA,
            pltpu.SemaphoreType.DMA,
            pltpu.SemaphoreType.DMA,
            pltpu.SemaphoreType.DMA,
        ],
    )(_sc_body)
    return run(latent, labels3d, centers)


def kernel(latent, labels, centers):
    labels3d = labels.astype(jnp.int32).reshape(_NW, _NCH, _CH)
    partials = _center_loss_partials(latent, labels3d, centers)
    return jnp.sum(partials) / jnp.float32(_B)


# speedup vs baseline: 1.0238x; 1.0238x over previous
import functools

import jax
import jax.numpy as jnp
from jax import lax
from jax.experimental import pallas as pl
from jax.experimental.pallas import tpu as pltpu
from jax.experimental.pallas import tpu_sc as plsc

_B = 16384
_D = 128
_C = 1000
_NC = 2
_NS = 16
_NW = _NC * _NS
_RPW = _B // _NW
_CH = 64
_NCH = _RPW // _CH
_NBUF = 4
_LANES = 16
_JV = _D // _LANES
_CPT = 64


def _sc_body(latent_hbm, labels_hbm, centers_hbm, out_hbm,
             lab_v, buf0, buf1, buf2, buf3, stage_v, res_v, neg_sh,
             fill_sem0, fill_sem1, fill_sem2, fill_sem3,
             add_sem0, add_sem1, add_sem2, add_sem3, out_sem):
    sid = lax.axis_index("s")
    wid = sid * _NC + lax.axis_index("c")
    pltpu.sync_copy(labels_hbm.at[wid], lab_v)

    base = jnp.minimum(sid * _CPT, _C - _CPT)
    pltpu.sync_copy(centers_hbm.at[pl.ds(base, _CPT)], stage_v)

    @plsc.parallel_loop(0, _CPT, 1)
    def _neg_loop(r):
        for j in range(_JV):
            stage_v[r, pl.ds(j * _LANES, _LANES)] = (
                -stage_v[r, pl.ds(j * _LANES, _LANES)])

    pltpu.sync_copy(stage_v, neg_sh.at[pl.ds(base, _CPT)])

    bufs = (buf0, buf1, buf2, buf3)
    fill_sems = (fill_sem0, fill_sem1, fill_sem2, fill_sem3)
    add_sems = (add_sem0, add_sem1, add_sem2, add_sem3)

    def fill(ch):
        b = ch % _NBUF
        row0 = wid * _RPW + ch * _CH
        return pltpu.async_copy(
            latent_hbm.at[pl.ds(row0, _CH)], bufs[b], fill_sems[b])

    def gather_add(ch):
        b = ch % _NBUF
        return pltpu.async_copy(
            neg_sh.at[lab_v.at[ch]], bufs[b], add_sems[b], add=True)

    accs = tuple(jnp.zeros((_LANES,), jnp.float32) for _ in range(_JV))

    fills = {}
    adds = {}
    for ch in range(_NBUF - 1):
        fills[ch] = fill(ch)
    plsc.subcore_barrier()
    fills[_NBUF - 1] = fill(_NBUF - 1)
    for ch in range(_NBUF - 2):
        fills[ch].wait()
        adds[ch] = gather_add(ch)
    for ch in range(_NCH):
        b = ch % _NBUF
        adds[ch].wait()
        buf = bufs[b]

        @plsc.parallel_loop(0, _CH, 1, unroll=4, carry=accs)
        def row_loop(r, acc_in):
            new = []
            for j in range(_JV):
                d = buf[r, pl.ds(j * _LANES, _LANES)]
                new.append(acc_in[j] + d * d)
            return tuple(new)

        accs = row_loop
        if ch + _NBUF < _NCH:
            fills[ch + _NBUF] = fill(ch + _NBUF)
        if ch + _NBUF - 2 < _NCH:
            fills[ch + _NBUF - 2].wait()
            adds[ch + _NBUF - 2] = gather_add(ch + _NBUF - 2)

    total = accs[0]
    for j in range(1, _JV):
        total = total + accs[j]
    res_v[...] = total
    pltpu.async_copy(res_v, out_hbm.at[wid], out_sem).wait()


@jax.jit
def _center_loss_partials(latent, labels3d, centers):
    mesh = plsc.VectorSubcoreMesh(core_axis_name="c", subcore_axis_name="s")
    run = functools.partial(
        pl.kernel,
        out_type=jax.ShapeDtypeStruct((_NW, _LANES), jnp.float32),
        mesh=mesh,
        scratch_types=[
            pltpu.VMEM((_NCH, _CH), jnp.int32),
            pltpu.VMEM((_CH, _D), jnp.float32),
            pltpu.VMEM((_CH, _D), jnp.float32),
            pltpu.VMEM((_CH, _D), jnp.float32),
            pltpu.VMEM((_CH, _D), jnp.float32),
            pltpu.VMEM((_CPT, _D), jnp.float32),
            pltpu.VMEM((_LANES,), jnp.float32),
            pltpu.VMEM_SHARED((_C, _D), jnp.float32),
            pltpu.SemaphoreType.DMA,
            pltpu.SemaphoreType.DMA,
            pltpu.SemaphoreType.DMA,
            pltpu.SemaphoreType.DMA,
            pltpu.SemaphoreType.DMA,
            pltpu.SemaphoreType.DMA,
            pltpu.SemaphoreType.DMA,
            pltpu.SemaphoreType.DMA,
            pltpu.SemaphoreType.DMA,
        ],
    )(_sc_body)
    return run(latent, labels3d, centers)


def kernel(latent, labels, centers):
    labels3d = labels.astype(jnp.int32).reshape(_NW, _NCH, _CH)
    partials = _center_loss_partials(latent, labels3d, centers)
    return jnp.sum(partials) / jnp.float32(_B)
